# baseline (speedup 1.0000x reference)
import functools

import jax
import jax.numpy as jnp
from jax.experimental import pallas as pl
from jax.experimental.pallas import tpu as pltpu

DIN = 128
DOUT = 128
BM = 256


def _prep_kernel(h_ref, wcat_ref, p_ref, r_ref):
    x = jax.lax.dot_general(
        h_ref[...], wcat_ref[...], (((1,), (0,)), ((), ())),
        precision=jax.lax.Precision.HIGHEST,
        preferred_element_type=jnp.float32,
    )
    p_ref[...] = x[:, : 3 * DOUT].astype(jnp.bfloat16)
    r_ref[...] = x[:, 3 * DOUT :]


def _main_kernel(a0_ref, a1_ref, a2_ref, p_ref, r_ref, h_ref, gu_ref, out_ref,
                 *, scale):
    p = p_ref[...]
    acc = jnp.dot(a0_ref[...].astype(jnp.bfloat16), p[:, :DOUT],
                  preferred_element_type=jnp.float32)
    acc += jnp.dot(a1_ref[...].astype(jnp.bfloat16), p[:, DOUT : 2 * DOUT],
                   preferred_element_type=jnp.float32)
    acc += jnp.dot(a2_ref[...].astype(jnp.bfloat16), p[:, 2 * DOUT :],
                   preferred_element_type=jnp.float32)
    r = r_ref[...]
    update = r[:, :DOUT] + scale * acc
    gate_pre = jax.lax.dot_general(
        update, gu_ref[...], (((1,), (0,)), ((), ())),
        precision=jax.lax.Precision.HIGHEST,
        preferred_element_type=jnp.float32,
    ) + r[:, DOUT:]
    g = jax.nn.sigmoid(gate_pre)
    h = h_ref[...]
    out_ref[...] = jnp.tanh(update) * g + h * (1.0 - g)


def kernel(H, adj_rel_0, adj_rel_1, adj_rel_2, W0, W_rel_0, W_rel_1, W_rel_2,
           gate_weight):
    n = H.shape[0]
    scale = 1.0 / (n - 1)
    wcat = jnp.concatenate(
        [W_rel_0, W_rel_1, W_rel_2, W0, gate_weight[DIN:]], axis=1)
    gu = gate_weight[:DIN]

    p, r = pl.pallas_call(
        _prep_kernel,
        out_shape=[
            jax.ShapeDtypeStruct((n, 3 * DOUT), jnp.bfloat16),
            jax.ShapeDtypeStruct((n, 2 * DOUT), jnp.float32),
        ],
    )(H, wcat)

    out = pl.pallas_call(
        functools.partial(_main_kernel, scale=scale),
        grid=(n // BM,),
        in_specs=[
            pl.BlockSpec((BM, n), lambda i: (i, 0)),
            pl.BlockSpec((BM, n), lambda i: (i, 0)),
            pl.BlockSpec((BM, n), lambda i: (i, 0)),
            pl.BlockSpec((n, 3 * DOUT), lambda i: (0, 0)),
            pl.BlockSpec((BM, 2 * DOUT), lambda i: (i, 0)),
            pl.BlockSpec((BM, DIN), lambda i: (i, 0)),
            pl.BlockSpec((DIN, DOUT), lambda i: (0, 0)),
        ],
        out_specs=pl.BlockSpec((BM, DOUT), lambda i: (i, 0)),
        out_shape=jax.ShapeDtypeStruct((n, DOUT), jnp.float32),
        compiler_params=pltpu.CompilerParams(
            dimension_semantics=("parallel",)),
    )(adj_rel_0, adj_rel_1, adj_rel_2, p, r, H, gu)
    return out
